# baseline (speedup 1.0000x reference)
import functools

import jax
import jax.numpy as jnp
import numpy as np
from jax import lax
from jax.experimental import pallas as pl
from jax.experimental.pallas import tpu as pltpu
from jax.experimental.pallas import tpu_sc as plsc

_B = 1024
_D = 128
_P = 4
_NNEG = 64
_KI = 72
_KS = 80
_NC = 2
_NS = 16
_NW = _NC * _NS
_BW = _B // _NW
_INV_T = 1.0 / 0.07
_LANES = 16
_DW = _D // 2
_WC = _DW // _LANES
_NBUF = 4

_PERM = np.array([32 * c + 2 * j + t
                  for c in range(_WC) for t in range(2) for j in range(_LANES)],
                 dtype=np.int32)


def _rsqrt16(x):
    xi = plsc.bitcast(x, jnp.int32)
    y = plsc.bitcast(jnp.int32(0x5F3759DF) - (xi >> 1), jnp.float32)
    for _ in range(3):
        y = y * (1.5 - 0.5 * x * y * y)
    return y


def _sc_sims_body(anch_hbm, idx_hbm, emb_hbm, out_hbm,
                  idx_v, anch_v, rb0, rb1, rb2, rb3,
                  dots_v, norms_v, sims_v, sem0, sem1, sem2, sem3):
    wid = lax.axis_index("s") * _NC + lax.axis_index("c")
    base = wid * _BW
    pltpu.sync_copy(idx_hbm.at[pl.ds(base, _BW)], idx_v)
    pltpu.sync_copy(anch_hbm.at[pl.ds(base, _BW)], anch_v)
    ones = jnp.ones((_LANES,), jnp.float32)
    dots_v[pl.ds(64, _LANES)] = ones
    norms_v[pl.ds(64, _LANES)] = ones
    lane15 = lax.iota(jnp.int32, _LANES) == (_LANES - 1)

    bufs = (rb0, rb1, rb2, rb3)
    sems = (sem0, sem1, sem2, sem3)

    def issue_gather(a, buf, sem):
        def body(g, c):
            vec = idx_v[a, pl.ds(g * _LANES, _LANES)]
            for j in range(_LANES):
                pltpu.async_copy(emb_hbm.at[vec[j]], buf.at[g * _LANES + j], sem)
            return c
        lax.fori_loop(0, 4, body, 0)
        tail = idx_v[a, pl.ds(56, _LANES)]
        for j in range(8, _LANES):
            pltpu.async_copy(emb_hbm.at[tail[j]], buf.at[56 + j], sem)

    def drain_gather(buf, sem):
        pltpu.make_async_copy(emb_hbm.at[pl.ds(0, _KI)], buf, sem).wait()

    for b in range(_NBUF - 1):
        issue_gather(b, bufs[b], sems[b])

    def compute_anchor(a, rows_v):
        ach = [anch_v[a, pl.ds(c * _LANES, _LANES)] for c in range(2 * _WC)]
        an = ach[0] * ach[0]
        for c in range(1, 2 * _WC):
            an = an + ach[c] * ach[c]
        a_scale = _rsqrt16(jnp.full((_LANES,), jnp.sum(an), jnp.float32)) * _INV_T

        @plsc.parallel_loop(0, _KI, 1, unroll=4)
        def row_body(r):
            fs = []
            for c in range(_WC):
                w = rows_v[r, pl.ds(c * _LANES, _LANES)]
                fs.append(plsc.bitcast(w << 16, jnp.float32))
                fs.append(plsc.bitcast(w & jnp.int32(-65536), jnp.float32))
            dterms = [fs[k] * ach[k] for k in range(2 * _WC)]
            nterms = [fs[k] * fs[k] for k in range(2 * _WC)]
            while len(dterms) > 1:
                dterms = [dterms[i] + dterms[i + 1] for i in range(0, len(dterms), 2)]
                nterms = [nterms[i] + nterms[i + 1] for i in range(0, len(nterms), 2)]
            accd, accn = dterms[0], nterms[0]
            ridx = jnp.full((_LANES,), r, jnp.int32)
            plsc.store_scatter(dots_v, [ridx], plsc.cumsum(accd), mask=lane15)
            plsc.store_scatter(norms_v, [ridx], plsc.cumsum(accn), mask=lane15)

        for g in range(_KS // _LANES):
            dv = dots_v[pl.ds(g * _LANES, _LANES)]
            nv = norms_v[pl.ds(g * _LANES, _LANES)]
            sims_v[a, pl.ds(g * _LANES, _LANES)] = dv * _rsqrt16(nv) * a_scale

    def outer(g, carry):
        for b in range(_NBUF):
            a = g * _NBUF + b
            drain_gather(bufs[b], sems[b])
            nxt = a + (_NBUF - 1)
            nb = (b + _NBUF - 1) % _NBUF

            @pl.when(nxt < _BW)
            def _():
                issue_gather(nxt, bufs[nb], sems[nb])

            compute_anchor(a, bufs[b])
        return carry

    lax.fori_loop(0, _BW // _NBUF, outer, 0)
    pltpu.sync_copy(sims_v, out_hbm.at[pl.ds(base, _BW)])


_sc_sims = pl.kernel(
    _sc_sims_body,
    out_type=jax.ShapeDtypeStruct((_B, _KS), jnp.float32),
    mesh=plsc.VectorSubcoreMesh(core_axis_name="c", subcore_axis_name="s",
                                num_cores=_NC, num_subcores=_NS),
    compiler_params=pltpu.CompilerParams(needs_layout_passes=False),
    scratch_types=[
        pltpu.VMEM((_BW, _KI), jnp.int32),
        pltpu.VMEM((_BW, _D), jnp.float32),
        pltpu.VMEM((_KI, _DW), jnp.int32),
        pltpu.VMEM((_KI, _DW), jnp.int32),
        pltpu.VMEM((_KI, _DW), jnp.int32),
        pltpu.VMEM((_KI, _DW), jnp.int32),
        pltpu.VMEM((_KS,), jnp.float32),
        pltpu.VMEM((_KS,), jnp.float32),
        pltpu.VMEM((_BW, _KS), jnp.float32),
        pltpu.SemaphoreType.DMA,
        pltpu.SemaphoreType.DMA,
        pltpu.SemaphoreType.DMA,
        pltpu.SemaphoreType.DMA,
    ],
)


def _tc_loss_body(sims_ref, out_ref):
    s = sims_ref[:]
    col = lax.broadcasted_iota(jnp.int32, (_B, _KS), 1)
    is_neg = col < _NNEG
    is_pos = (col >= _NNEG) & (col < _NNEG + _P)
    m = jnp.max(jnp.where(is_neg, s, jnp.float32(-3.0e38)), axis=1, keepdims=True)
    ssum = jnp.sum(jnp.where(is_neg, jnp.exp(s - m), 0.0), axis=1, keepdims=True)
    big = jnp.maximum(m, s)
    lse = big + jnp.log(jnp.exp(s - big) + ssum * jnp.exp(m - big))
    out_ref[0, 0] = jnp.sum(jnp.where(is_pos, lse - s, 0.0)) / (_B * _P)


_tc_loss = pl.pallas_call(
    _tc_loss_body,
    out_shape=jax.ShapeDtypeStruct((1, 1), jnp.float32),
    out_specs=pl.BlockSpec(memory_space=pltpu.SMEM),
)


def kernel(anchor_embeddings, positive_indices, negative_indices, all_embeddings):
    pad = jnp.zeros((_B, _KI - _NNEG - _P), jnp.int32)
    idx = jnp.concatenate(
        [negative_indices.astype(jnp.int32), positive_indices.astype(jnp.int32), pad],
        axis=1)
    packed = jax.lax.bitcast_convert_type(
        all_embeddings.astype(jnp.bfloat16).reshape(-1, _DW, 2), jnp.int32)
    sims = _sc_sims(anchor_embeddings[:, _PERM], idx, packed)
    return _tc_loss(sims)[0, 0]


# speedup vs baseline: 14.3914x; 14.3914x over previous
import functools

import jax
import jax.numpy as jnp
from jax import lax
from jax.experimental import pallas as pl
from jax.experimental.pallas import tpu as pltpu
from jax.experimental.pallas import tpu_sc as plsc

_B = 1024
_D = 128
_P = 4
_NNEG = 64
_KI = 72
_KG = 68
_KS = 80
_NC = 2
_NS = 16
_NW = _NC * _NS
_BW = _B // _NW
_INV_T = 1.0 / 0.07
_LANES = 16
_DC = _D // _LANES
_NBUF = 4


def _rsqrt16(x):
    xi = plsc.bitcast(x, jnp.int32)
    y = plsc.bitcast(jnp.int32(0x5F3759DF) - (xi >> 1), jnp.float32)
    for _ in range(3):
        y = y * (1.5 - 0.5 * x * y * y)
    return y


def _sc_sims_body(anch_hbm, idx_hbm, emb_hbm, out_hbm,
                  idx_v, anch_v, rb0, rb1, rb2, rb3,
                  dots_v, norms_v, sims_v, sem0, sem1, sem2, sem3):
    wid = lax.axis_index("s") * _NC + lax.axis_index("c")
    base = wid * _BW
    pltpu.sync_copy(idx_hbm.at[pl.ds(base, _BW)], idx_v)
    pltpu.sync_copy(anch_hbm.at[pl.ds(base, _BW)], anch_v)
    ones = jnp.ones((_LANES,), jnp.float32)
    dots_v[pl.ds(64, _LANES)] = ones
    norms_v[pl.ds(64, _LANES)] = ones
    lane15 = lax.iota(jnp.int32, _LANES) == (_LANES - 1)

    bufs = (rb0, rb1, rb2, rb3)
    sems = (sem0, sem1, sem2, sem3)

    for b in range(_NBUF - 1):
        pltpu.async_copy(emb_hbm.at[idx_v.at[b, pl.ds(0, _KG)]], bufs[b], sems[b])

    def compute_anchor(a, rows_v):
        ach = [anch_v[a, pl.ds(c * _LANES, _LANES)] for c in range(_DC)]
        an = ach[0] * ach[0]
        for c in range(1, _DC):
            an = an + ach[c] * ach[c]
        a_scale = _rsqrt16(jnp.full((_LANES,), jnp.sum(an), jnp.float32)) * _INV_T

        @plsc.parallel_loop(0, _KG, 1, unroll=4)
        def row_body(r):
            vs = [rows_v[r, pl.ds(c * _LANES, _LANES)] for c in range(_DC)]
            dterms = [vs[c] * ach[c] for c in range(_DC)]
            nterms = [vs[c] * vs[c] for c in range(_DC)]
            while len(dterms) > 1:
                dterms = [dterms[i] + dterms[i + 1] for i in range(0, len(dterms), 2)]
                nterms = [nterms[i] + nterms[i + 1] for i in range(0, len(nterms), 2)]
            accd, accn = dterms[0], nterms[0]
            ridx = jnp.full((_LANES,), r, jnp.int32)
            plsc.store_scatter(dots_v, [ridx], plsc.cumsum(accd), mask=lane15)
            plsc.store_scatter(norms_v, [ridx], plsc.cumsum(accn), mask=lane15)

        for g in range(_KS // _LANES):
            dv = dots_v[pl.ds(g * _LANES, _LANES)]
            nv = norms_v[pl.ds(g * _LANES, _LANES)]
            sims_v[a, pl.ds(g * _LANES, _LANES)] = dv * _rsqrt16(nv) * a_scale

    def outer(g, carry):
        for b in range(_NBUF):
            a = g * _NBUF + b
            pltpu.make_async_copy(emb_hbm.at[idx_v.at[a, pl.ds(0, _KG)]], bufs[b], sems[b]).wait()
            nxt = a + (_NBUF - 1)
            nb = (b + _NBUF - 1) % _NBUF

            @pl.when(nxt < _BW)
            def _():
                pltpu.async_copy(emb_hbm.at[idx_v.at[nxt, pl.ds(0, _KG)]], bufs[nb], sems[nb])

            compute_anchor(a, bufs[b])
        return carry

    lax.fori_loop(0, _BW // _NBUF, outer, 0)
    pltpu.sync_copy(sims_v, out_hbm.at[pl.ds(base, _BW)])


_sc_sims = pl.kernel(
    _sc_sims_body,
    out_type=jax.ShapeDtypeStruct((_B, _KS), jnp.float32),
    mesh=plsc.VectorSubcoreMesh(core_axis_name="c", subcore_axis_name="s",
                                num_cores=_NC, num_subcores=_NS),
    compiler_params=pltpu.CompilerParams(needs_layout_passes=False),
    scratch_types=[
        pltpu.VMEM((_BW, _KI), jnp.int32),
        pltpu.VMEM((_BW, _D), jnp.float32),
        pltpu.VMEM((_KG, _D), jnp.float32),
        pltpu.VMEM((_KG, _D), jnp.float32),
        pltpu.VMEM((_KG, _D), jnp.float32),
        pltpu.VMEM((_KG, _D), jnp.float32),
        pltpu.VMEM((_KS,), jnp.float32),
        pltpu.VMEM((_KS,), jnp.float32),
        pltpu.VMEM((_BW, _KS), jnp.float32),
        pltpu.SemaphoreType.DMA,
        pltpu.SemaphoreType.DMA,
        pltpu.SemaphoreType.DMA,
        pltpu.SemaphoreType.DMA,
    ],
)


def _tc_loss_body(sims_ref, out_ref):
    s = sims_ref[:]
    col = lax.broadcasted_iota(jnp.int32, (_B, _KS), 1)
    is_neg = col < _NNEG
    is_pos = (col >= _NNEG) & (col < _NNEG + _P)
    m = jnp.max(jnp.where(is_neg, s, jnp.float32(-3.0e38)), axis=1, keepdims=True)
    ssum = jnp.sum(jnp.where(is_neg, jnp.exp(s - m), 0.0), axis=1, keepdims=True)
    big = jnp.maximum(m, s)
    lse = big + jnp.log(jnp.exp(s - big) + ssum * jnp.exp(m - big))
    out_ref[0, 0] = jnp.sum(jnp.where(is_pos, lse - s, 0.0)) / (_B * _P)


_tc_loss = pl.pallas_call(
    _tc_loss_body,
    out_shape=jax.ShapeDtypeStruct((1, 1), jnp.float32),
    out_specs=pl.BlockSpec(memory_space=pltpu.SMEM),
)


def kernel(anchor_embeddings, positive_indices, negative_indices, all_embeddings):
    pad = jnp.zeros((_B, _KI - _NNEG - _P), jnp.int32)
    idx = jnp.concatenate(
        [negative_indices.astype(jnp.int32), positive_indices.astype(jnp.int32), pad],
        axis=1)
    sims = _sc_sims(anchor_embeddings, idx, all_embeddings)
    return _tc_loss(sims)[0, 0]


# speedup vs baseline: 14.4449x; 1.0037x over previous
import jax
import jax.numpy as jnp
from jax import lax
from jax.experimental import pallas as pl
from jax.experimental.pallas import tpu as pltpu
from jax.experimental.pallas import tpu_sc as plsc

_B = 1024
_D = 128
_P = 4
_NNEG = 64
_KI = 72
_KG = 68
_KS = 80
_NC = 2
_NS = 16
_NW = _NC * _NS
_BW = _B // _NW
_INV_T = 1.0 / 0.07
_LANES = 16
_DC = _D // _LANES
_NBUF = 4


def _rsqrt16(x):
    xi = plsc.bitcast(x, jnp.int32)
    y = plsc.bitcast(jnp.int32(0x5F3759DF) - (xi >> 1), jnp.float32)
    for _ in range(3):
        y = y * (1.5 - 0.5 * x * y * y)
    return y


def _sc_sims_body(anch_hbm, idx_hbm, emb_hbm, out_hbm,
                  idx_v, anch_v, rb0, rb1, rb2, rb3,
                  dots_v, norms_v, sims_v, sem0, sem1, sem2, sem3):
    wid = lax.axis_index("s") * _NC + lax.axis_index("c")
    base = wid * _BW
    pltpu.sync_copy(idx_hbm.at[pl.ds(base, _BW)], idx_v)
    pltpu.sync_copy(anch_hbm.at[pl.ds(base, _BW)], anch_v)
    ones = jnp.ones((_LANES,), jnp.float32)
    dots_v[pl.ds(64, _LANES)] = ones
    norms_v[pl.ds(64, _LANES)] = ones
    lane15 = lax.iota(jnp.int32, _LANES) == (_LANES - 1)

    bufs = (rb0, rb1, rb2, rb3)
    sems = (sem0, sem1, sem2, sem3)

    for b in range(_NBUF - 1):
        pltpu.async_copy(emb_hbm.at[idx_v.at[b, pl.ds(0, _KG)]], bufs[b], sems[b])

    def compute_anchor(a, rows_v):
        ach = [anch_v[a, pl.ds(c * _LANES, _LANES)] for c in range(_DC)]
        an = ach[0] * ach[0]
        for c in range(1, _DC):
            an = an + ach[c] * ach[c]
        a_scale = _rsqrt16(jnp.full((_LANES,), jnp.sum(an), jnp.float32)) * _INV_T

        @plsc.parallel_loop(0, _KG, 1, unroll=4)
        def row_body(r):
            vs = [rows_v[r, pl.ds(c * _LANES, _LANES)] for c in range(_DC)]
            dterms = [vs[c] * ach[c] for c in range(_DC)]
            nterms = [vs[c] * vs[c] for c in range(_DC)]
            while len(dterms) > 1:
                dterms = [dterms[i] + dterms[i + 1] for i in range(0, len(dterms), 2)]
                nterms = [nterms[i] + nterms[i + 1] for i in range(0, len(nterms), 2)]
            accd, accn = dterms[0], nterms[0]
            ridx = jnp.full((_LANES,), r, jnp.int32)
            plsc.store_scatter(dots_v, [ridx], plsc.cumsum(accd), mask=lane15)
            plsc.store_scatter(norms_v, [ridx], plsc.cumsum(accn), mask=lane15)

        for g in range(_KS // _LANES):
            dv = dots_v[pl.ds(g * _LANES, _LANES)]
            nv = norms_v[pl.ds(g * _LANES, _LANES)]
            sims_v[a, pl.ds(g * _LANES, _LANES)] = dv * _rsqrt16(nv) * a_scale

    def outer(g, carry):
        for b in range(_NBUF):
            a = g * _NBUF + b
            pltpu.make_async_copy(emb_hbm.at[idx_v.at[a, pl.ds(0, _KG)]], bufs[b], sems[b]).wait()
            nxt = a + (_NBUF - 1)
            nb = (b + _NBUF - 1) % _NBUF

            @pl.when(nxt < _BW)
            def _():
                pltpu.async_copy(emb_hbm.at[idx_v.at[nxt, pl.ds(0, _KG)]], bufs[nb], sems[nb])

            compute_anchor(a, bufs[b])
        return carry

    lax.fori_loop(0, _BW // _NBUF, outer, 0)
    pltpu.sync_copy(sims_v, out_hbm.at[pl.ds(base, _BW)])


_sc_sims = pl.kernel(
    _sc_sims_body,
    out_type=jax.ShapeDtypeStruct((_B, _KS), jnp.float32),
    mesh=plsc.VectorSubcoreMesh(core_axis_name="c", subcore_axis_name="s",
                                num_cores=_NC, num_subcores=_NS),
    compiler_params=pltpu.CompilerParams(needs_layout_passes=False),
    scratch_types=[
        pltpu.VMEM((_BW, _KI), jnp.int32),
        pltpu.VMEM((_BW, _D), jnp.float32),
        pltpu.VMEM((_KG, _D), jnp.float32),
        pltpu.VMEM((_KG, _D), jnp.float32),
        pltpu.VMEM((_KG, _D), jnp.float32),
        pltpu.VMEM((_KG, _D), jnp.float32),
        pltpu.VMEM((_KS,), jnp.float32),
        pltpu.VMEM((_KS,), jnp.float32),
        pltpu.VMEM((_BW, _KS), jnp.float32),
        pltpu.SemaphoreType.DMA,
        pltpu.SemaphoreType.DMA,
        pltpu.SemaphoreType.DMA,
        pltpu.SemaphoreType.DMA,
    ],
)


def _tc_loss_body(sims_ref, out_ref):
    s = sims_ref[:]
    col = lax.broadcasted_iota(jnp.int32, (_B, _KS), 1)
    is_neg = col < _NNEG
    is_pos = (col >= _NNEG) & (col < _NNEG + _P)
    m = jnp.max(jnp.where(is_neg, s, jnp.float32(-3.0e38)), axis=1, keepdims=True)
    ssum = jnp.sum(jnp.where(is_neg, jnp.exp(s - m), 0.0), axis=1, keepdims=True)
    big = jnp.maximum(m, s)
    lse = big + jnp.log(jnp.exp(s - big) + ssum * jnp.exp(m - big))
    out_ref[0, 0] = jnp.sum(jnp.where(is_pos, lse - s, 0.0)) / (_B * _P)


_tc_loss = pl.pallas_call(
    _tc_loss_body,
    out_shape=jax.ShapeDtypeStruct((1, 1), jnp.float32),
    out_specs=pl.BlockSpec(memory_space=pltpu.SMEM),
)


def kernel(anchor_embeddings, positive_indices, negative_indices, all_embeddings):
    pad = jnp.zeros((_B, _KI - _NNEG - _P), jnp.int32)
    idx = jnp.concatenate(
        [negative_indices.astype(jnp.int32), positive_indices.astype(jnp.int32), pad],
        axis=1)
    sims = _sc_sims(anchor_embeddings, idx, all_embeddings)
    return _tc_loss(sims)[0, 0]
